# baseline (speedup 1.0000x reference)
import functools

import jax
import jax.numpy as jnp
from jax import lax
from jax.experimental import pallas as pl
from jax.experimental.pallas import tpu as pltpu
from jax.experimental.pallas import tpu_sc as plsc

N = 10000
E = 320000
D_IN = 128
DH = 128
DE = 16
L = 4
EPS = 1e-5

NC = 2
NS = 16
NW = NC * NS
EPW = E // NW
CHUNK = 40
NCHUNK = EPW // CHUNK
NZ = N // CHUNK


def _fold_body(x_r, node_W_r, node_b_r, edge_W_r, We_r, be_r, edge_b_r,
               h0_r, M_r, c_r):
    h0_r[...] = (jnp.dot(x_r[...], node_W_r[...],
                         preferred_element_type=jnp.float32)
                 + node_b_r[...][None, :])
    ew = edge_W_r[...]
    eb = edge_b_r[...]
    for l in range(L):
        Wl = We_r[l]
        M_r[l] = jnp.dot(ew, Wl, preferred_element_type=jnp.float32)
        c_r[l] = jnp.dot(eb[None, :], Wl,
                         preferred_element_type=jnp.float32)[0] + be_r[l]


def _fold_call(x, node_W, node_b, edge_W, We, be, edge_b):
    return pl.pallas_call(
        _fold_body,
        out_shape=(
            jax.ShapeDtypeStruct((N, DH), jnp.float32),
            jax.ShapeDtypeStruct((L, DE, DH), jnp.float32),
            jax.ShapeDtypeStruct((L, DH), jnp.float32),
        ),
    )(x, node_W, node_b, edge_W, We, be, edge_b)


EBLK = 8000


def _efeat_body(ea_r, M_r, c_r, *out_rs):
    a = ea_r[...]
    for l in range(L):
        out_rs[l][...] = (jnp.dot(a, M_r[l], preferred_element_type=jnp.float32)
                          + c_r[l][None, :])


def _efeat_call(edge_attr, M, c):
    nblk = E // EBLK
    return pl.pallas_call(
        _efeat_body,
        grid=(nblk,),
        in_specs=[
            pl.BlockSpec((EBLK, DE), lambda i: (i, 0)),
            pl.BlockSpec((L, DE, DH), lambda i: (0, 0, 0)),
            pl.BlockSpec((L, DH), lambda i: (0, 0)),
        ],
        out_specs=tuple(pl.BlockSpec((EBLK, DH), lambda i: (i, 0))
                        for _ in range(L)),
        out_shape=tuple(jax.ShapeDtypeStruct((E, DH), jnp.float32)
                        for _ in range(L)),
    )(edge_attr, M, c)


PBLK = 2000


def _pre_body(h_r, Wk_r, bk_r, Wq_r, bq_r, Wv_r, bv_r, Ws_r, bs_r,
              k_r, qv_r, skip_r):
    h = h_r[...]
    k_r[...] = jnp.dot(h, Wk_r[...], preferred_element_type=jnp.float32) + bk_r[...][None, :]
    q = jnp.dot(h, Wq_r[...], preferred_element_type=jnp.float32) + bq_r[...][None, :]
    v = jnp.dot(h, Wv_r[...], preferred_element_type=jnp.float32) + bv_r[...][None, :]
    qv_r[...] = jnp.concatenate([q, v], axis=1)
    skip_r[...] = jnp.dot(h, Ws_r[...], preferred_element_type=jnp.float32) + bs_r[...][None, :]


def _pre_call(h, Wk, bk, Wq, bq, Wv, bv, Ws, bs):
    nblk = N // PBLK
    wspec = pl.BlockSpec((DH, DH), lambda i: (0, 0))
    bspec = pl.BlockSpec((DH,), lambda i: (0,))
    return pl.pallas_call(
        _pre_body,
        grid=(nblk,),
        in_specs=[pl.BlockSpec((PBLK, DH), lambda i: (i, 0)),
                  wspec, bspec, wspec, bspec, wspec, bspec, wspec, bspec],
        out_specs=(pl.BlockSpec((PBLK, DH), lambda i: (i, 0)),
                   pl.BlockSpec((PBLK, 2 * DH), lambda i: (i, 0)),
                   pl.BlockSpec((PBLK, DH), lambda i: (i, 0))),
        out_shape=(jax.ShapeDtypeStruct((N, DH), jnp.float32),
                   jax.ShapeDtypeStruct((N, 2 * DH), jnp.float32),
                   jax.ShapeDtypeStruct((N, DH), jnp.float32)),
    )(h, Wk, bk, Wq, bq, Wv, bv, Ws, bs)


def _post_body(h_r, skip_r, parts_r, gamma_r, beta_r, out_r):
    n = skip_r[...] + parts_r[0] + parts_r[1]
    mu = jnp.mean(n, axis=0)
    d = n - mu[None, :]
    var = jnp.mean(d * d, axis=0)
    bn = gamma_r[...][None, :] * d * lax.rsqrt(var + EPS)[None, :] + beta_r[...][None, :]
    out_r[...] = (h_r[...] + jnp.maximum(bn, 0.0)) * 0.5


def _post_call(h, skip, parts, gamma, beta):
    return pl.pallas_call(
        _post_body,
        out_shape=jax.ShapeDtypeStruct((N, DH), jnp.float32),
    )(h, skip, parts, gamma, beta)


def _head_body(h_r, W_r, b_r, out_r):
    out_r[...] = (jnp.dot(h_r[...], W_r[...], preferred_element_type=jnp.float32)
                  + b_r[...][None, :])


def _head_call(h, W, b):
    return pl.pallas_call(
        _head_body,
        out_shape=jax.ShapeDtypeStruct((N, DH), jnp.float32),
    )(h, W, b)


def _edge_body(src_hbm, dst_hbm, k_hbm, qv_hbm, e_hbm, out_hbm,
               agg, srcv0, srcv1, dstv0, dstv1, dsts0, dsts1,
               kv0, kv1, qv0, qv1, ev0, ev1,
               sem_k, sem_qv, sem_e, sem_src, sem_dst, sem_sc):
    cid = lax.axis_index("c")
    sid = lax.axis_index("s")
    wid = cid * NS + sid
    ebase = wid * EPW

    srcb = [srcv0, srcv1]
    dstb = [dstv0, dstv1]
    dstsb = [dsts0, dsts1]
    kb = [kv0, kv1]
    qvb = [qv0, qv1]
    eb = [ev0, ev1]

    @plsc.parallel_loop(0, CHUNK)
    def _(r):
        for g in range(DH // 16):
            kv0[r, pl.ds(g * 16, 16)] = jnp.zeros((16,), jnp.float32)

    def zloop(i, _):
        j = sid + i * NS

        @pl.when(j < NZ)
        def _():
            pltpu.sync_copy(kv0, agg.at[pl.ds(j * CHUNK, CHUNK)])

        return 0

    lax.fori_loop(0, (NZ + NS - 1) // NS, zloop, 0)
    plsc.subcore_barrier()

    pltpu.sync_copy(src_hbm.at[pl.ds(ebase, CHUNK)], srcv0)
    pltpu.sync_copy(dst_hbm.at[pl.ds(ebase, CHUNK)], dstv0)
    pltpu.async_copy(k_hbm.at[dstv0], kv0, sem_k)
    pltpu.async_copy(qv_hbm.at[srcv0], qv0, sem_qv)
    pltpu.async_copy(e_hbm.at[pl.ds(ebase, CHUNK)], ev0, sem_e)
    pltpu.async_copy(src_hbm.at[pl.ds(ebase + CHUNK, CHUNK)], srcv1, sem_src)
    pltpu.async_copy(dst_hbm.at[pl.ds(ebase + CHUNK, CHUNK)], dstv1, sem_dst)

    def pipe(i, _):
        for b in range(2):
            j = 2 * i + b
            nb = 1 - b
            pltpu.make_async_copy(k_hbm.at[dstb[b]], kb[b], sem_k).wait()
            pltpu.make_async_copy(qv_hbm.at[srcb[b]], qvb[b], sem_qv).wait()
            pltpu.make_async_copy(e_hbm.at[pl.ds(0, CHUNK)], eb[b],
                                  sem_e).wait()

            @pl.when(j > 0)
            def _():
                pltpu.make_async_copy(kb[nb], agg.at[dstsb[nb]],
                                      sem_sc).wait()

            @pl.when(j + 1 < NCHUNK)
            def _():
                pltpu.make_async_copy(src_hbm.at[pl.ds(0, CHUNK)], srcb[nb],
                                      sem_src).wait()
                pltpu.make_async_copy(dst_hbm.at[pl.ds(0, CHUNK)], dstb[nb],
                                      sem_dst).wait()
                pltpu.async_copy(k_hbm.at[dstb[nb]], kb[nb], sem_k)
                pltpu.async_copy(qv_hbm.at[srcb[nb]], qvb[nb], sem_qv)
                pltpu.async_copy(
                    e_hbm.at[pl.ds(ebase + (j + 1) * CHUNK, CHUNK)],
                    eb[nb], sem_e)

            for o in (0, 16, 24):
                dstsb[b][pl.ds(o, 16)] = dstb[b][pl.ds(o, 16)]

            @plsc.parallel_loop(0, CHUNK, unroll=8)
            def _(c):
                for g in range(DH // 16):
                    kk = kb[b][c, pl.ds(g * 16, 16)]
                    qq = qvb[b][c, pl.ds(g * 16, 16)]
                    vv = qvb[b][c, pl.ds(DH + g * 16, 16)]
                    ee = eb[b][c, pl.ds(g * 16, 16)]
                    t = kk + qq + ee
                    s = 1.0 / (1.0 + jnp.exp(-t))
                    kb[b][c, pl.ds(g * 16, 16)] = s * vv

            pltpu.async_copy(kb[b], agg.at[dstsb[b]], sem_sc, add=True)

            @pl.when(j + 2 < NCHUNK)
            def _():
                base2 = ebase + (j + 2) * CHUNK
                pltpu.async_copy(src_hbm.at[pl.ds(base2, CHUNK)], srcb[b],
                                 sem_src)
                pltpu.async_copy(dst_hbm.at[pl.ds(base2, CHUNK)], dstb[b],
                                 sem_dst)

        return 0

    lax.fori_loop(0, NCHUNK // 2, pipe, 0)
    pltpu.make_async_copy(kb[1], agg.at[dstsb[1]], sem_sc).wait()
    plsc.subcore_barrier()

    def wloop(i, _):
        j = sid + i * NS

        @pl.when(j < NZ)
        def _():
            pltpu.sync_copy(agg.at[pl.ds(j * CHUNK, CHUNK)],
                            out_hbm.at[cid, pl.ds(j * CHUNK, CHUNK)])

        return 0

    lax.fori_loop(0, (NZ + NS - 1) // NS, wloop, 0)


@functools.partial(
    pl.kernel,
    mesh=plsc.VectorSubcoreMesh(core_axis_name="c", subcore_axis_name="s"),
    out_type=jax.ShapeDtypeStruct((NC, N, DH), jnp.float32),
    scratch_types=[
        pltpu.VMEM_SHARED((N, DH), jnp.float32),
        pltpu.VMEM((CHUNK,), jnp.int32),
        pltpu.VMEM((CHUNK,), jnp.int32),
        pltpu.VMEM((CHUNK,), jnp.int32),
        pltpu.VMEM((CHUNK,), jnp.int32),
        pltpu.VMEM((CHUNK,), jnp.int32),
        pltpu.VMEM((CHUNK,), jnp.int32),
        pltpu.VMEM((CHUNK, DH), jnp.float32),
        pltpu.VMEM((CHUNK, DH), jnp.float32),
        pltpu.VMEM((CHUNK, 2 * DH), jnp.float32),
        pltpu.VMEM((CHUNK, 2 * DH), jnp.float32),
        pltpu.VMEM((CHUNK, DH), jnp.float32),
        pltpu.VMEM((CHUNK, DH), jnp.float32),
        pltpu.SemaphoreType.DMA,
        pltpu.SemaphoreType.DMA,
        pltpu.SemaphoreType.DMA,
        pltpu.SemaphoreType.DMA,
        pltpu.SemaphoreType.DMA,
        pltpu.SemaphoreType.DMA,
    ],
)
def _edge_call(src_hbm, dst_hbm, k_hbm, qv_hbm, e_hbm, out_hbm,
               agg, srcv0, srcv1, dstv0, dstv1, dsts0, dsts1,
               kv0, kv1, qv0, qv1, ev0, ev1,
               sem_k, sem_qv, sem_e, sem_src, sem_dst, sem_sc):
    _edge_body(src_hbm, dst_hbm, k_hbm, qv_hbm, e_hbm, out_hbm,
               agg, srcv0, srcv1, dstv0, dstv1, dsts0, dsts1,
               kv0, kv1, qv0, qv1, ev0, ev1,
               sem_k, sem_qv, sem_e, sem_src, sem_dst, sem_sc)


def kernel(x, edge_index, edge_attr, node_W, node_b, edge_W, edge_b,
           Wk, bk, Wq, bq, Wv, bv, Ws, bs, We, be, gamma, beta,
           head_W, head_b):
    src = edge_index[0]
    dst = edge_index[1]
    h, M, c = _fold_call(x, node_W, node_b, edge_W, We, be, edge_b)
    e_feats = _efeat_call(edge_attr, M, c)
    for l in range(L):
        k, qv, skip = _pre_call(h, Wk[l], bk[l], Wq[l], bq[l],
                                Wv[l], bv[l], Ws[l], bs[l])
        parts = _edge_call(src, dst, k, qv, e_feats[l])
        h = _post_call(h, skip, parts, gamma[l], beta[l])
    return _head_call(h, head_W, head_b)


# speedup vs baseline: 1.2401x; 1.2401x over previous
import functools

import jax
import jax.numpy as jnp
from jax import lax
from jax.experimental import pallas as pl
from jax.experimental.pallas import tpu as pltpu
from jax.experimental.pallas import tpu_sc as plsc

N = 10000
E = 320000
D_IN = 128
DH = 128
DE = 16
L = 4
EPS = 1e-5

NC = 2
NS = 16
NW = NC * NS
EPW = E // NW
CHUNK = 40
NCHUNK = EPW // CHUNK
NZ = N // CHUNK


def _fold_body(x_r, node_W_r, node_b_r, edge_W_r, We_r, be_r, edge_b_r,
               h0_r, M_r, c_r):
    h0_r[...] = (jnp.dot(x_r[...], node_W_r[...],
                         preferred_element_type=jnp.float32)
                 + node_b_r[...][None, :])
    ew = edge_W_r[...]
    eb = edge_b_r[...]
    for l in range(L):
        Wl = We_r[l]
        M_r[l] = jnp.dot(ew, Wl, preferred_element_type=jnp.float32)
        c_r[l] = jnp.dot(eb[None, :], Wl,
                         preferred_element_type=jnp.float32)[0] + be_r[l]


def _fold_call(x, node_W, node_b, edge_W, We, be, edge_b):
    return pl.pallas_call(
        _fold_body,
        out_shape=(
            jax.ShapeDtypeStruct((N, DH), jnp.float32),
            jax.ShapeDtypeStruct((L, DE, DH), jnp.float32),
            jax.ShapeDtypeStruct((L, DH), jnp.float32),
        ),
    )(x, node_W, node_b, edge_W, We, be, edge_b)


EBLK = 8000


def _efeat_body(ea_r, M_r, c_r, *out_rs):
    a = ea_r[...]
    for l in range(L):
        out_rs[l][...] = (jnp.dot(a, M_r[l], preferred_element_type=jnp.float32)
                          + c_r[l][None, :])


def _efeat_call(edge_attr, M, c):
    nblk = E // EBLK
    return pl.pallas_call(
        _efeat_body,
        grid=(nblk,),
        in_specs=[
            pl.BlockSpec((EBLK, DE), lambda i: (i, 0)),
            pl.BlockSpec((L, DE, DH), lambda i: (0, 0, 0)),
            pl.BlockSpec((L, DH), lambda i: (0, 0)),
        ],
        out_specs=tuple(pl.BlockSpec((EBLK, DH), lambda i: (i, 0))
                        for _ in range(L)),
        out_shape=tuple(jax.ShapeDtypeStruct((E, DH), jnp.float32)
                        for _ in range(L)),
    )(edge_attr, M, c)


PBLK = 2000


def _pre_body(h_r, Wk_r, bk_r, Wq_r, bq_r, Wv_r, bv_r, Ws_r, bs_r,
              k_r, qv_r, skip_r):
    h = h_r[...]
    k_r[...] = jnp.dot(h, Wk_r[...], preferred_element_type=jnp.float32) + bk_r[...][None, :]
    q = jnp.dot(h, Wq_r[...], preferred_element_type=jnp.float32) + bq_r[...][None, :]
    v = jnp.dot(h, Wv_r[...], preferred_element_type=jnp.float32) + bv_r[...][None, :]
    qv_r[...] = jnp.concatenate([q, v], axis=1)
    skip_r[...] = jnp.dot(h, Ws_r[...], preferred_element_type=jnp.float32) + bs_r[...][None, :]


def _pre_call(h, Wk, bk, Wq, bq, Wv, bv, Ws, bs):
    nblk = N // PBLK
    wspec = pl.BlockSpec((DH, DH), lambda i: (0, 0))
    bspec = pl.BlockSpec((DH,), lambda i: (0,))
    return pl.pallas_call(
        _pre_body,
        grid=(nblk,),
        in_specs=[pl.BlockSpec((PBLK, DH), lambda i: (i, 0)),
                  wspec, bspec, wspec, bspec, wspec, bspec, wspec, bspec],
        out_specs=(pl.BlockSpec((PBLK, DH), lambda i: (i, 0)),
                   pl.BlockSpec((PBLK, 2 * DH), lambda i: (i, 0)),
                   pl.BlockSpec((PBLK, DH), lambda i: (i, 0))),
        out_shape=(jax.ShapeDtypeStruct((N, DH), jnp.float32),
                   jax.ShapeDtypeStruct((N, 2 * DH), jnp.float32),
                   jax.ShapeDtypeStruct((N, DH), jnp.float32)),
    )(h, Wk, bk, Wq, bq, Wv, bv, Ws, bs)


def _post_body(h_r, skip_r, parts_r, gamma_r, beta_r, out_r):
    n = skip_r[...] + parts_r[0] + parts_r[1]
    mu = jnp.mean(n, axis=0)
    d = n - mu[None, :]
    var = jnp.mean(d * d, axis=0)
    bn = gamma_r[...][None, :] * d * lax.rsqrt(var + EPS)[None, :] + beta_r[...][None, :]
    out_r[...] = (h_r[...] + jnp.maximum(bn, 0.0)) * 0.5


def _post_call(h, skip, parts, gamma, beta):
    return pl.pallas_call(
        _post_body,
        out_shape=jax.ShapeDtypeStruct((N, DH), jnp.float32),
    )(h, skip, parts, gamma, beta)


def _head_body(h_r, W_r, b_r, out_r):
    out_r[...] = (jnp.dot(h_r[...], W_r[...], preferred_element_type=jnp.float32)
                  + b_r[...][None, :])


def _head_call(h, W, b):
    return pl.pallas_call(
        _head_body,
        out_shape=jax.ShapeDtypeStruct((N, DH), jnp.float32),
    )(h, W, b)


def _edge_body(src_hbm, dst_hbm, k_hbm, qv_hbm, e_hbm, out_hbm,
               agg, srcv0, srcv1, dstv0, dstv1, dsts0, dsts1,
               kv0, kv1, qv0, qv1, ev0, ev1,
               sem_k, sem_qv, sem_e, sem_src, sem_dst, sem_sc):
    cid = lax.axis_index("c")
    sid = lax.axis_index("s")
    wid = cid * NS + sid
    ebase = wid * EPW

    srcb = [srcv0, srcv1]
    dstb = [dstv0, dstv1]
    dstsb = [dsts0, dsts1]
    kb = [kv0, kv1]
    qvb = [qv0, qv1]
    eb = [ev0, ev1]

    @plsc.parallel_loop(0, CHUNK)
    def _(r):
        for g in range(DH // 16):
            kv0[r, pl.ds(g * 16, 16)] = jnp.zeros((16,), jnp.float32)

    def zloop(i, _):
        j = sid + i * NS

        @pl.when(j < NZ)
        def _():
            pltpu.sync_copy(kv0, agg.at[pl.ds(j * CHUNK, CHUNK)])

        return 0

    lax.fori_loop(0, (NZ + NS - 1) // NS, zloop, 0)
    plsc.subcore_barrier()

    pltpu.sync_copy(src_hbm.at[pl.ds(ebase, CHUNK)], srcv0)
    pltpu.sync_copy(dst_hbm.at[pl.ds(ebase, CHUNK)], dstv0)
    pltpu.async_copy(k_hbm.at[dstv0], kv0, sem_k)
    pltpu.async_copy(qv_hbm.at[srcv0], qv0, sem_qv)
    pltpu.async_copy(e_hbm.at[pl.ds(ebase, CHUNK)], ev0, sem_e)
    pltpu.async_copy(src_hbm.at[pl.ds(ebase + CHUNK, CHUNK)], srcv1, sem_src)
    pltpu.async_copy(dst_hbm.at[pl.ds(ebase + CHUNK, CHUNK)], dstv1, sem_dst)

    def pipe(i, _):
        for b in range(2):
            j = 2 * i + b
            nb = 1 - b
            pltpu.make_async_copy(k_hbm.at[dstb[b]], kb[b], sem_k).wait()
            pltpu.make_async_copy(qv_hbm.at[srcb[b]], qvb[b], sem_qv).wait()
            pltpu.make_async_copy(e_hbm.at[pl.ds(0, CHUNK)], eb[b],
                                  sem_e).wait()

            @pl.when(j > 0)
            def _():
                pltpu.make_async_copy(kb[nb], agg.at[dstsb[nb]],
                                      sem_sc).wait()

            @pl.when(j + 1 < NCHUNK)
            def _():
                pltpu.make_async_copy(src_hbm.at[pl.ds(0, CHUNK)], srcb[nb],
                                      sem_src).wait()
                pltpu.make_async_copy(dst_hbm.at[pl.ds(0, CHUNK)], dstb[nb],
                                      sem_dst).wait()
                pltpu.async_copy(k_hbm.at[dstb[nb]], kb[nb], sem_k)
                pltpu.async_copy(qv_hbm.at[srcb[nb]], qvb[nb], sem_qv)
                pltpu.async_copy(
                    e_hbm.at[pl.ds(ebase + (j + 1) * CHUNK, CHUNK)],
                    eb[nb], sem_e)

            for o in (0, 16, 24):
                dstsb[b][pl.ds(o, 16)] = dstb[b][pl.ds(o, 16)]

            @plsc.parallel_loop(0, CHUNK, unroll=2)
            def _(c):
                for g in range(DH // 16):
                    kk = kb[b][c, pl.ds(g * 16, 16)]
                    qq = qvb[b][c, pl.ds(g * 16, 16)]
                    vv = qvb[b][c, pl.ds(DH + g * 16, 16)]
                    ee = eb[b][c, pl.ds(g * 16, 16)]
                    t = kk + qq + ee
                    s = 1.0 / (1.0 + jnp.exp(-t))
                    kb[b][c, pl.ds(g * 16, 16)] = s * vv

            pltpu.async_copy(kb[b], agg.at[dstsb[b]], sem_sc, add=True)

            @pl.when(j + 2 < NCHUNK)
            def _():
                base2 = ebase + (j + 2) * CHUNK
                pltpu.async_copy(src_hbm.at[pl.ds(base2, CHUNK)], srcb[b],
                                 sem_src)
                pltpu.async_copy(dst_hbm.at[pl.ds(base2, CHUNK)], dstb[b],
                                 sem_dst)

        return 0

    lax.fori_loop(0, NCHUNK // 2, pipe, 0)
    pltpu.make_async_copy(kb[1], agg.at[dstsb[1]], sem_sc).wait()
    plsc.subcore_barrier()

    def wloop(i, _):
        j = sid + i * NS

        @pl.when(j < NZ)
        def _():
            pltpu.sync_copy(agg.at[pl.ds(j * CHUNK, CHUNK)],
                            out_hbm.at[cid, pl.ds(j * CHUNK, CHUNK)])

        return 0

    lax.fori_loop(0, (NZ + NS - 1) // NS, wloop, 0)


@functools.partial(
    pl.kernel,
    mesh=plsc.VectorSubcoreMesh(core_axis_name="c", subcore_axis_name="s"),
    out_type=jax.ShapeDtypeStruct((NC, N, DH), jnp.float32),
    scratch_types=[
        pltpu.VMEM_SHARED((N, DH), jnp.float32),
        pltpu.VMEM((CHUNK,), jnp.int32),
        pltpu.VMEM((CHUNK,), jnp.int32),
        pltpu.VMEM((CHUNK,), jnp.int32),
        pltpu.VMEM((CHUNK,), jnp.int32),
        pltpu.VMEM((CHUNK,), jnp.int32),
        pltpu.VMEM((CHUNK,), jnp.int32),
        pltpu.VMEM((CHUNK, DH), jnp.float32),
        pltpu.VMEM((CHUNK, DH), jnp.float32),
        pltpu.VMEM((CHUNK, 2 * DH), jnp.float32),
        pltpu.VMEM((CHUNK, 2 * DH), jnp.float32),
        pltpu.VMEM((CHUNK, DH), jnp.float32),
        pltpu.VMEM((CHUNK, DH), jnp.float32),
        pltpu.SemaphoreType.DMA,
        pltpu.SemaphoreType.DMA,
        pltpu.SemaphoreType.DMA,
        pltpu.SemaphoreType.DMA,
        pltpu.SemaphoreType.DMA,
        pltpu.SemaphoreType.DMA,
    ],
)
def _edge_call(src_hbm, dst_hbm, k_hbm, qv_hbm, e_hbm, out_hbm,
               agg, srcv0, srcv1, dstv0, dstv1, dsts0, dsts1,
               kv0, kv1, qv0, qv1, ev0, ev1,
               sem_k, sem_qv, sem_e, sem_src, sem_dst, sem_sc):
    _edge_body(src_hbm, dst_hbm, k_hbm, qv_hbm, e_hbm, out_hbm,
               agg, srcv0, srcv1, dstv0, dstv1, dsts0, dsts1,
               kv0, kv1, qv0, qv1, ev0, ev1,
               sem_k, sem_qv, sem_e, sem_src, sem_dst, sem_sc)


def kernel(x, edge_index, edge_attr, node_W, node_b, edge_W, edge_b,
           Wk, bk, Wq, bq, Wv, bv, Ws, bs, We, be, gamma, beta,
           head_W, head_b):
    src = edge_index[0]
    dst = edge_index[1]
    h, M, c = _fold_call(x, node_W, node_b, edge_W, We, be, edge_b)
    e_feats = _efeat_call(edge_attr, M, c)
    for l in range(L):
        k, qv, skip = _pre_call(h, Wk[l], bk[l], Wq[l], bq[l],
                                Wv[l], bv[l], Ws[l], bs[l])
        parts = _edge_call(src, dst, k, qv, e_feats[l])
        h = _post_call(h, skip, parts, gamma[l], beta[l])
    return _head_call(h, head_W, head_b)


# speedup vs baseline: 1.3659x; 1.1015x over previous
import functools

import jax
import jax.numpy as jnp
from jax import lax
from jax.experimental import pallas as pl
from jax.experimental.pallas import tpu as pltpu
from jax.experimental.pallas import tpu_sc as plsc

N = 10000
E = 320000
D_IN = 128
DH = 128
DE = 16
L = 4
EPS = 1e-5

NC = 2
NS = 16
NW = NC * NS
EPW = E // NW
CHUNK = 40
NCHUNK = EPW // CHUNK
NZ = N // CHUNK


def _fold_body(x_r, node_W_r, node_b_r, edge_W_r, We_r, be_r, edge_b_r,
               h0_r, M_r, c_r):
    h0_r[...] = (jnp.dot(x_r[...], node_W_r[...],
                         preferred_element_type=jnp.float32)
                 + node_b_r[...][None, :])
    ew = edge_W_r[...]
    eb = edge_b_r[...]
    for l in range(L):
        Wl = We_r[l]
        M_r[l] = jnp.dot(ew, Wl, preferred_element_type=jnp.float32)
        c_r[l] = jnp.dot(eb[None, :], Wl,
                         preferred_element_type=jnp.float32)[0] + be_r[l]


def _fold_call(x, node_W, node_b, edge_W, We, be, edge_b):
    return pl.pallas_call(
        _fold_body,
        out_shape=(
            jax.ShapeDtypeStruct((N, DH), jnp.float32),
            jax.ShapeDtypeStruct((L, DE, DH), jnp.float32),
            jax.ShapeDtypeStruct((L, DH), jnp.float32),
        ),
    )(x, node_W, node_b, edge_W, We, be, edge_b)


EBLK = 8000


def _efeat_body(ea_r, M_r, c_r, *out_rs):
    a = ea_r[...]
    for l in range(L):
        out_rs[l][...] = (jnp.dot(a, M_r[l], preferred_element_type=jnp.float32)
                          + c_r[l][None, :])


def _efeat_call(edge_attr, M, c):
    nblk = E // EBLK
    return pl.pallas_call(
        _efeat_body,
        grid=(nblk,),
        in_specs=[
            pl.BlockSpec((EBLK, DE), lambda i: (i, 0)),
            pl.BlockSpec((L, DE, DH), lambda i: (0, 0, 0)),
            pl.BlockSpec((L, DH), lambda i: (0, 0)),
        ],
        out_specs=tuple(pl.BlockSpec((EBLK, DH), lambda i: (i, 0))
                        for _ in range(L)),
        out_shape=tuple(jax.ShapeDtypeStruct((E, DH), jnp.float32)
                        for _ in range(L)),
    )(edge_attr, M, c)


PBLK = 2000


def _pre_body(h_r, Wk_r, bk_r, Wq_r, bq_r, Wv_r, bv_r, Ws_r, bs_r,
              k_r, qv_r, skip_r):
    h = h_r[...]
    k_r[...] = jnp.dot(h, Wk_r[...], preferred_element_type=jnp.float32) + bk_r[...][None, :]
    q = jnp.dot(h, Wq_r[...], preferred_element_type=jnp.float32) + bq_r[...][None, :]
    v = jnp.dot(h, Wv_r[...], preferred_element_type=jnp.float32) + bv_r[...][None, :]
    qv_r[...] = jnp.concatenate([q, v], axis=1)
    skip_r[...] = jnp.dot(h, Ws_r[...], preferred_element_type=jnp.float32) + bs_r[...][None, :]


def _pre_call(h, Wk, bk, Wq, bq, Wv, bv, Ws, bs):
    nblk = N // PBLK
    wspec = pl.BlockSpec((DH, DH), lambda i: (0, 0))
    bspec = pl.BlockSpec((DH,), lambda i: (0,))
    return pl.pallas_call(
        _pre_body,
        grid=(nblk,),
        in_specs=[pl.BlockSpec((PBLK, DH), lambda i: (i, 0)),
                  wspec, bspec, wspec, bspec, wspec, bspec, wspec, bspec],
        out_specs=(pl.BlockSpec((PBLK, DH), lambda i: (i, 0)),
                   pl.BlockSpec((PBLK, 2 * DH), lambda i: (i, 0)),
                   pl.BlockSpec((PBLK, DH), lambda i: (i, 0))),
        out_shape=(jax.ShapeDtypeStruct((N, DH), jnp.float32),
                   jax.ShapeDtypeStruct((N, 2 * DH), jnp.float32),
                   jax.ShapeDtypeStruct((N, DH), jnp.float32)),
    )(h, Wk, bk, Wq, bq, Wv, bv, Ws, bs)


def _post_body(h_r, skip_r, parts_r, gamma_r, beta_r, out_r):
    n = skip_r[...] + parts_r[0] + parts_r[1]
    mu = jnp.mean(n, axis=0)
    d = n - mu[None, :]
    var = jnp.mean(d * d, axis=0)
    bn = gamma_r[...][None, :] * d * lax.rsqrt(var + EPS)[None, :] + beta_r[...][None, :]
    out_r[...] = (h_r[...] + jnp.maximum(bn, 0.0)) * 0.5


def _post_call(h, skip, parts, gamma, beta):
    return pl.pallas_call(
        _post_body,
        out_shape=jax.ShapeDtypeStruct((N, DH), jnp.float32),
    )(h, skip, parts, gamma, beta)


def _head_body(h_r, W_r, b_r, out_r):
    out_r[...] = (jnp.dot(h_r[...], W_r[...], preferred_element_type=jnp.float32)
                  + b_r[...][None, :])


def _head_call(h, W, b):
    return pl.pallas_call(
        _head_body,
        out_shape=jax.ShapeDtypeStruct((N, DH), jnp.float32),
    )(h, W, b)


def _edge_body(src_hbm, dst_hbm, k_hbm, qv_hbm, e_hbm, out_hbm,
               agg, srcv0, srcv1, dstv0, dstv1, dsts0, dsts1,
               kv0, kv1, qv0, qv1, ev0, ev1,
               sem_k, sem_qv, sem_e, sem_src, sem_dst, sem_sc):
    cid = lax.axis_index("c")
    sid = lax.axis_index("s")
    wid = cid * NS + sid
    ebase = wid * EPW

    srcb = [srcv0, srcv1]
    dstb = [dstv0, dstv1]
    dstsb = [dsts0, dsts1]
    kb = [kv0, kv1]
    qvb = [qv0, qv1]
    eb = [ev0, ev1]

    @plsc.parallel_loop(0, CHUNK)
    def _(r):
        for g in range(DH // 16):
            kv0[r, pl.ds(g * 16, 16)] = jnp.zeros((16,), jnp.float32)

    def zloop(i, _):
        j = sid + i * NS

        @pl.when(j < NZ)
        def _():
            pltpu.sync_copy(kv0, agg.at[pl.ds(j * CHUNK, CHUNK)])

        return 0

    lax.fori_loop(0, (NZ + NS - 1) // NS, zloop, 0)
    plsc.subcore_barrier()

    pltpu.sync_copy(src_hbm.at[pl.ds(ebase, CHUNK)], srcv0)
    pltpu.sync_copy(dst_hbm.at[pl.ds(ebase, CHUNK)], dstv0)
    pltpu.async_copy(k_hbm.at[dstv0], kv0, sem_k)
    pltpu.async_copy(qv_hbm.at[srcv0], qv0, sem_qv)
    pltpu.async_copy(e_hbm.at[pl.ds(ebase, CHUNK)], ev0, sem_e)
    pltpu.async_copy(src_hbm.at[pl.ds(ebase + CHUNK, CHUNK)], srcv1, sem_src)
    pltpu.async_copy(dst_hbm.at[pl.ds(ebase + CHUNK, CHUNK)], dstv1, sem_dst)

    def pipe(i, _):
        for b in range(2):
            j = 2 * i + b
            nb = 1 - b
            pltpu.make_async_copy(k_hbm.at[dstb[b]], kb[b], sem_k).wait()
            pltpu.make_async_copy(qv_hbm.at[srcb[b]], qvb[b], sem_qv).wait()
            pltpu.make_async_copy(e_hbm.at[pl.ds(0, CHUNK)], eb[b],
                                  sem_e).wait()

            @pl.when(j > 0)
            def _():
                pltpu.make_async_copy(kb[nb], agg.at[dstsb[nb]],
                                      sem_sc).wait()

            @pl.when(j + 1 < NCHUNK)
            def _():
                pltpu.make_async_copy(src_hbm.at[pl.ds(0, CHUNK)], srcb[nb],
                                      sem_src).wait()
                pltpu.make_async_copy(dst_hbm.at[pl.ds(0, CHUNK)], dstb[nb],
                                      sem_dst).wait()
                pltpu.async_copy(k_hbm.at[dstb[nb]], kb[nb], sem_k)
                pltpu.async_copy(qv_hbm.at[srcb[nb]], qvb[nb], sem_qv)
                pltpu.async_copy(
                    e_hbm.at[pl.ds(ebase + (j + 1) * CHUNK, CHUNK)],
                    eb[nb], sem_e)

            for o in (0, 16, 24):
                dstsb[b][pl.ds(o, 16)] = dstb[b][pl.ds(o, 16)]

            @plsc.parallel_loop(0, CHUNK)
            def _(c):
                for g in range(DH // 16):
                    kk = kb[b][c, pl.ds(g * 16, 16)]
                    qq = qvb[b][c, pl.ds(g * 16, 16)]
                    vv = qvb[b][c, pl.ds(DH + g * 16, 16)]
                    ee = eb[b][c, pl.ds(g * 16, 16)]
                    t = kk + qq + ee
                    s = 1.0 / (1.0 + jnp.exp(-t))
                    kb[b][c, pl.ds(g * 16, 16)] = s * vv

            pltpu.async_copy(kb[b], agg.at[dstsb[b]], sem_sc, add=True)

            @pl.when(j + 2 < NCHUNK)
            def _():
                base2 = ebase + (j + 2) * CHUNK
                pltpu.async_copy(src_hbm.at[pl.ds(base2, CHUNK)], srcb[b],
                                 sem_src)
                pltpu.async_copy(dst_hbm.at[pl.ds(base2, CHUNK)], dstb[b],
                                 sem_dst)

        return 0

    lax.fori_loop(0, NCHUNK // 2, pipe, 0)
    pltpu.make_async_copy(kb[1], agg.at[dstsb[1]], sem_sc).wait()
    plsc.subcore_barrier()

    def wloop(i, _):
        j = sid + i * NS

        @pl.when(j < NZ)
        def _():
            pltpu.sync_copy(agg.at[pl.ds(j * CHUNK, CHUNK)],
                            out_hbm.at[cid, pl.ds(j * CHUNK, CHUNK)])

        return 0

    lax.fori_loop(0, (NZ + NS - 1) // NS, wloop, 0)


@functools.partial(
    pl.kernel,
    mesh=plsc.VectorSubcoreMesh(core_axis_name="c", subcore_axis_name="s"),
    out_type=jax.ShapeDtypeStruct((NC, N, DH), jnp.float32),
    scratch_types=[
        pltpu.VMEM_SHARED((N, DH), jnp.float32),
        pltpu.VMEM((CHUNK,), jnp.int32),
        pltpu.VMEM((CHUNK,), jnp.int32),
        pltpu.VMEM((CHUNK,), jnp.int32),
        pltpu.VMEM((CHUNK,), jnp.int32),
        pltpu.VMEM((CHUNK,), jnp.int32),
        pltpu.VMEM((CHUNK,), jnp.int32),
        pltpu.VMEM((CHUNK, DH), jnp.float32),
        pltpu.VMEM((CHUNK, DH), jnp.float32),
        pltpu.VMEM((CHUNK, 2 * DH), jnp.float32),
        pltpu.VMEM((CHUNK, 2 * DH), jnp.float32),
        pltpu.VMEM((CHUNK, DH), jnp.float32),
        pltpu.VMEM((CHUNK, DH), jnp.float32),
        pltpu.SemaphoreType.DMA,
        pltpu.SemaphoreType.DMA,
        pltpu.SemaphoreType.DMA,
        pltpu.SemaphoreType.DMA,
        pltpu.SemaphoreType.DMA,
        pltpu.SemaphoreType.DMA,
    ],
)
def _edge_call(src_hbm, dst_hbm, k_hbm, qv_hbm, e_hbm, out_hbm,
               agg, srcv0, srcv1, dstv0, dstv1, dsts0, dsts1,
               kv0, kv1, qv0, qv1, ev0, ev1,
               sem_k, sem_qv, sem_e, sem_src, sem_dst, sem_sc):
    _edge_body(src_hbm, dst_hbm, k_hbm, qv_hbm, e_hbm, out_hbm,
               agg, srcv0, srcv1, dstv0, dstv1, dsts0, dsts1,
               kv0, kv1, qv0, qv1, ev0, ev1,
               sem_k, sem_qv, sem_e, sem_src, sem_dst, sem_sc)


def kernel(x, edge_index, edge_attr, node_W, node_b, edge_W, edge_b,
           Wk, bk, Wq, bq, Wv, bv, Ws, bs, We, be, gamma, beta,
           head_W, head_b):
    src = edge_index[0]
    dst = edge_index[1]
    h, M, c = _fold_call(x, node_W, node_b, edge_W, We, be, edge_b)
    e_feats = _efeat_call(edge_attr, M, c)
    for l in range(L):
        k, qv, skip = _pre_call(h, Wk[l], bk[l], Wq[l], bq[l],
                                Wv[l], bv[l], Ws[l], bs[l])
        parts = _edge_call(src, dst, k, qv, e_feats[l])
        h = _post_call(h, skip, parts, gamma[l], beta[l])
    return _head_call(h, head_W, head_b)


# speedup vs baseline: 1.3793x; 1.0098x over previous
import functools

import jax
import jax.numpy as jnp
from jax import lax
from jax.experimental import pallas as pl
from jax.experimental.pallas import tpu as pltpu
from jax.experimental.pallas import tpu_sc as plsc

N = 10000
E = 320000
D_IN = 128
DH = 128
DE = 16
L = 4
EPS = 1e-5

NC = 2
NS = 16
NW = NC * NS
EPW = E // NW
CHUNK = 40
NCHUNK = EPW // CHUNK
NZ = N // CHUNK


def _proj(h, W_r, b_r):
    return jnp.dot(h, W_r[...], preferred_element_type=jnp.float32) \
        + b_r[...][None, :]


def _fold_body(x_r, node_W_r, node_b_r, edge_W_r, We_r, be_r, edge_b_r,
               Wk_r, bk_r, Wq_r, bq_r, Wv_r, bv_r, Ws_r, bs_r,
               h0_r, M_r, c_r, k_r, qv_r, skip_r):
    h = (jnp.dot(x_r[...], node_W_r[...], preferred_element_type=jnp.float32)
         + node_b_r[...][None, :])
    h0_r[...] = h
    ew = edge_W_r[...]
    eb = edge_b_r[...]
    for l in range(L):
        Wl = We_r[l]
        M_r[l] = jnp.dot(ew, Wl, preferred_element_type=jnp.float32)
        c_r[l] = jnp.dot(eb[None, :], Wl,
                         preferred_element_type=jnp.float32)[0] + be_r[l]
    k_r[...] = _proj(h, Wk_r, bk_r)
    qv_r[...] = jnp.concatenate(
        [_proj(h, Wq_r, bq_r), _proj(h, Wv_r, bv_r)], axis=1)
    skip_r[...] = _proj(h, Ws_r, bs_r)


def _fold_call(x, node_W, node_b, edge_W, We, be, edge_b,
               Wk, bk, Wq, bq, Wv, bv, Ws, bs):
    return pl.pallas_call(
        _fold_body,
        out_shape=(
            jax.ShapeDtypeStruct((N, DH), jnp.float32),
            jax.ShapeDtypeStruct((L, DE, DH), jnp.float32),
            jax.ShapeDtypeStruct((L, DH), jnp.float32),
            jax.ShapeDtypeStruct((N, DH), jnp.float32),
            jax.ShapeDtypeStruct((N, 2 * DH), jnp.float32),
            jax.ShapeDtypeStruct((N, DH), jnp.float32),
        ),
    )(x, node_W, node_b, edge_W, We, be, edge_b,
      Wk, bk, Wq, bq, Wv, bv, Ws, bs)


EBLK = 8000


def _efeat_body(ea_r, M_r, c_r, *out_rs):
    a = ea_r[...]
    for l in range(L):
        out_rs[l][...] = (jnp.dot(a, M_r[l], preferred_element_type=jnp.float32)
                          + c_r[l][None, :])


def _efeat_call(edge_attr, M, c):
    nblk = E // EBLK
    return pl.pallas_call(
        _efeat_body,
        grid=(nblk,),
        in_specs=[
            pl.BlockSpec((EBLK, DE), lambda i: (i, 0)),
            pl.BlockSpec((L, DE, DH), lambda i: (0, 0, 0)),
            pl.BlockSpec((L, DH), lambda i: (0, 0)),
        ],
        out_specs=tuple(pl.BlockSpec((EBLK, DH), lambda i: (i, 0))
                        for _ in range(L)),
        out_shape=tuple(jax.ShapeDtypeStruct((E, DH), jnp.float32)
                        for _ in range(L)),
    )(edge_attr, M, c)


def _bn_merge(h_r, skip_r, parts_r, gamma_r, beta_r):
    n = skip_r[...] + parts_r[0] + parts_r[1]
    mu = jnp.mean(n, axis=0)
    d = n - mu[None, :]
    var = jnp.mean(d * d, axis=0)
    bn = gamma_r[...][None, :] * d * lax.rsqrt(var + EPS)[None, :] \
        + beta_r[...][None, :]
    return (h_r[...] + jnp.maximum(bn, 0.0)) * 0.5


def _postpre_body(h_r, skip_r, parts_r, gamma_r, beta_r,
                  Wk_r, bk_r, Wq_r, bq_r, Wv_r, bv_r, Ws_r, bs_r,
                  h_out, k_r, qv_r, skip_out):
    h = _bn_merge(h_r, skip_r, parts_r, gamma_r, beta_r)
    h_out[...] = h
    k_r[...] = _proj(h, Wk_r, bk_r)
    qv_r[...] = jnp.concatenate(
        [_proj(h, Wq_r, bq_r), _proj(h, Wv_r, bv_r)], axis=1)
    skip_out[...] = _proj(h, Ws_r, bs_r)


def _postpre_call(h, skip, parts, gamma, beta, Wk, bk, Wq, bq, Wv, bv, Ws, bs):
    return pl.pallas_call(
        _postpre_body,
        out_shape=(jax.ShapeDtypeStruct((N, DH), jnp.float32),
                   jax.ShapeDtypeStruct((N, DH), jnp.float32),
                   jax.ShapeDtypeStruct((N, 2 * DH), jnp.float32),
                   jax.ShapeDtypeStruct((N, DH), jnp.float32)),
    )(h, skip, parts, gamma, beta, Wk, bk, Wq, bq, Wv, bv, Ws, bs)


def _postlast_body(h_r, skip_r, parts_r, gamma_r, beta_r, W_r, b_r, out_r):
    h = _bn_merge(h_r, skip_r, parts_r, gamma_r, beta_r)
    out_r[...] = _proj(h, W_r, b_r)


def _postlast_call(h, skip, parts, gamma, beta, W, b):
    return pl.pallas_call(
        _postlast_body,
        out_shape=jax.ShapeDtypeStruct((N, DH), jnp.float32),
    )(h, skip, parts, gamma, beta, W, b)


def _edge_body(src_hbm, dst_hbm, k_hbm, qv_hbm, e_hbm, out_hbm,
               agg, srcv0, srcv1, dstv0, dstv1, dsts0, dsts1,
               kv0, kv1, qv0, qv1, ev0, ev1,
               sem_k, sem_qv, sem_e, sem_src, sem_dst, sem_sc):
    cid = lax.axis_index("c")
    sid = lax.axis_index("s")
    wid = cid * NS + sid
    ebase = wid * EPW

    srcb = [srcv0, srcv1]
    dstb = [dstv0, dstv1]
    dstsb = [dsts0, dsts1]
    kb = [kv0, kv1]
    qvb = [qv0, qv1]
    eb = [ev0, ev1]

    @plsc.parallel_loop(0, CHUNK)
    def _(r):
        for g in range(DH // 16):
            kv0[r, pl.ds(g * 16, 16)] = jnp.zeros((16,), jnp.float32)

    def zloop(i, _):
        j = sid + i * NS

        @pl.when(j < NZ)
        def _():
            pltpu.sync_copy(kv0, agg.at[pl.ds(j * CHUNK, CHUNK)])

        return 0

    lax.fori_loop(0, (NZ + NS - 1) // NS, zloop, 0)
    plsc.subcore_barrier()

    pltpu.sync_copy(src_hbm.at[pl.ds(ebase, CHUNK)], srcv0)
    pltpu.sync_copy(dst_hbm.at[pl.ds(ebase, CHUNK)], dstv0)
    pltpu.async_copy(k_hbm.at[dstv0], kv0, sem_k)
    pltpu.async_copy(qv_hbm.at[srcv0], qv0, sem_qv)
    pltpu.async_copy(e_hbm.at[pl.ds(ebase, CHUNK)], ev0, sem_e)
    pltpu.async_copy(src_hbm.at[pl.ds(ebase + CHUNK, CHUNK)], srcv1, sem_src)
    pltpu.async_copy(dst_hbm.at[pl.ds(ebase + CHUNK, CHUNK)], dstv1, sem_dst)

    def pipe(i, _):
        for b in range(2):
            j = 2 * i + b
            nb = 1 - b
            pltpu.make_async_copy(k_hbm.at[dstb[b]], kb[b], sem_k).wait()
            pltpu.make_async_copy(qv_hbm.at[srcb[b]], qvb[b], sem_qv).wait()
            pltpu.make_async_copy(e_hbm.at[pl.ds(0, CHUNK)], eb[b],
                                  sem_e).wait()

            @pl.when(j > 0)
            def _():
                pltpu.make_async_copy(kb[nb], agg.at[dstsb[nb]],
                                      sem_sc).wait()

            @pl.when(j + 1 < NCHUNK)
            def _():
                pltpu.make_async_copy(src_hbm.at[pl.ds(0, CHUNK)], srcb[nb],
                                      sem_src).wait()
                pltpu.make_async_copy(dst_hbm.at[pl.ds(0, CHUNK)], dstb[nb],
                                      sem_dst).wait()
                pltpu.async_copy(k_hbm.at[dstb[nb]], kb[nb], sem_k)
                pltpu.async_copy(qv_hbm.at[srcb[nb]], qvb[nb], sem_qv)
                pltpu.async_copy(
                    e_hbm.at[pl.ds(ebase + (j + 1) * CHUNK, CHUNK)],
                    eb[nb], sem_e)

            for o in (0, 16, 24):
                dstsb[b][pl.ds(o, 16)] = dstb[b][pl.ds(o, 16)]

            @plsc.parallel_loop(0, CHUNK)
            def _(c):
                for g in range(DH // 16):
                    kk = kb[b][c, pl.ds(g * 16, 16)]
                    qq = qvb[b][c, pl.ds(g * 16, 16)]
                    vv = qvb[b][c, pl.ds(DH + g * 16, 16)]
                    ee = eb[b][c, pl.ds(g * 16, 16)]
                    t = kk + qq + ee
                    s = 1.0 / (1.0 + jnp.exp(-t))
                    kb[b][c, pl.ds(g * 16, 16)] = s * vv

            pltpu.async_copy(kb[b], agg.at[dstsb[b]], sem_sc, add=True)

            @pl.when(j + 2 < NCHUNK)
            def _():
                base2 = ebase + (j + 2) * CHUNK
                pltpu.async_copy(src_hbm.at[pl.ds(base2, CHUNK)], srcb[b],
                                 sem_src)
                pltpu.async_copy(dst_hbm.at[pl.ds(base2, CHUNK)], dstb[b],
                                 sem_dst)

        return 0

    lax.fori_loop(0, NCHUNK // 2, pipe, 0)
    pltpu.make_async_copy(kb[1], agg.at[dstsb[1]], sem_sc).wait()
    plsc.subcore_barrier()

    def wloop(i, _):
        j = sid + i * NS

        @pl.when(j < NZ)
        def _():
            pltpu.sync_copy(agg.at[pl.ds(j * CHUNK, CHUNK)],
                            out_hbm.at[cid, pl.ds(j * CHUNK, CHUNK)])

        return 0

    lax.fori_loop(0, (NZ + NS - 1) // NS, wloop, 0)


@functools.partial(
    pl.kernel,
    mesh=plsc.VectorSubcoreMesh(core_axis_name="c", subcore_axis_name="s"),
    out_type=jax.ShapeDtypeStruct((NC, N, DH), jnp.float32),
    scratch_types=[
        pltpu.VMEM_SHARED((N, DH), jnp.float32),
        pltpu.VMEM((CHUNK,), jnp.int32),
        pltpu.VMEM((CHUNK,), jnp.int32),
        pltpu.VMEM((CHUNK,), jnp.int32),
        pltpu.VMEM((CHUNK,), jnp.int32),
        pltpu.VMEM((CHUNK,), jnp.int32),
        pltpu.VMEM((CHUNK,), jnp.int32),
        pltpu.VMEM((CHUNK, DH), jnp.float32),
        pltpu.VMEM((CHUNK, DH), jnp.float32),
        pltpu.VMEM((CHUNK, 2 * DH), jnp.float32),
        pltpu.VMEM((CHUNK, 2 * DH), jnp.float32),
        pltpu.VMEM((CHUNK, DH), jnp.float32),
        pltpu.VMEM((CHUNK, DH), jnp.float32),
        pltpu.SemaphoreType.DMA,
        pltpu.SemaphoreType.DMA,
        pltpu.SemaphoreType.DMA,
        pltpu.SemaphoreType.DMA,
        pltpu.SemaphoreType.DMA,
        pltpu.SemaphoreType.DMA,
    ],
)
def _edge_call(src_hbm, dst_hbm, k_hbm, qv_hbm, e_hbm, out_hbm,
               agg, srcv0, srcv1, dstv0, dstv1, dsts0, dsts1,
               kv0, kv1, qv0, qv1, ev0, ev1,
               sem_k, sem_qv, sem_e, sem_src, sem_dst, sem_sc):
    _edge_body(src_hbm, dst_hbm, k_hbm, qv_hbm, e_hbm, out_hbm,
               agg, srcv0, srcv1, dstv0, dstv1, dsts0, dsts1,
               kv0, kv1, qv0, qv1, ev0, ev1,
               sem_k, sem_qv, sem_e, sem_src, sem_dst, sem_sc)


def kernel(x, edge_index, edge_attr, node_W, node_b, edge_W, edge_b,
           Wk, bk, Wq, bq, Wv, bv, Ws, bs, We, be, gamma, beta,
           head_W, head_b):
    src = edge_index[0]
    dst = edge_index[1]
    h, M, c, k, qv, skip = _fold_call(x, node_W, node_b, edge_W, We, be,
                                      edge_b, Wk[0], bk[0], Wq[0], bq[0],
                                      Wv[0], bv[0], Ws[0], bs[0])
    e_feats = _efeat_call(edge_attr, M, c)
    for l in range(L):
        parts = _edge_call(src, dst, k, qv, e_feats[l])
        if l < L - 1:
            h, k, qv, skip = _postpre_call(
                h, skip, parts, gamma[l], beta[l], Wk[l + 1], bk[l + 1],
                Wq[l + 1], bq[l + 1], Wv[l + 1], bv[l + 1],
                Ws[l + 1], bs[l + 1])
        else:
            out = _postlast_call(h, skip, parts, gamma[l], beta[l],
                                 head_W, head_b)
    return out
